# baseline (speedup 1.0000x reference)
import jax
import jax.numpy as jnp
from jax.experimental import pallas as pl
from jax.experimental.pallas import tpu as pltpu

N = 320000
D = 128
S = 1024
BN = 512
NB = N // BN
W = 4
OB = 128
NEP = S // OB


def _fused_kernel(bounds_ref, x_ref, pos_ref, ids_ref, wx_ref, wp_ref,
                  out_ref, acc_ref):
    i = pl.program_id(0)

    @pl.when(i == 0)
    def _init():
        acc_ref[...] = jnp.zeros_like(acc_ref)

    @pl.when(i < NB)
    def _compute():
        h = jnp.dot(x_ref[...], wx_ref[...],
                    preferred_element_type=jnp.float32)
        h += jnp.dot(pos_ref[...], wp_ref[...],
                     preferred_element_type=jnp.float32)

        ids = ids_ref[0]
        s_lo = bounds_ref[i, 0]
        s_hi = bounds_ref[i, 1]

        base = jnp.minimum(s_lo, S - W)
        win = jnp.concatenate(
            [jnp.max(jnp.where(ids == base + k, h, 0.0), axis=0,
                     keepdims=True) for k in range(W)], axis=0)
        cur = acc_ref[pl.ds(base, W), :]
        acc_ref[pl.ds(base, W), :] = jnp.maximum(cur, win)

        @pl.when(s_hi >= base + W)
        def _resid():
            def body(s, _):
                col = jnp.max(jnp.where(ids == s, h, 0.0), axis=0,
                              keepdims=True)
                cur = acc_ref[pl.ds(s, 1), :]
                acc_ref[pl.ds(s, 1), :] = jnp.maximum(cur, col)
                return 0

            jax.lax.fori_loop(base + W, s_hi + 1, body, 0, unroll=False)

    @pl.when(i >= NB)
    def _epilogue():
        out_ref[...] = acc_ref[pl.ds((i - NB) * OB, OB), :]


def kernel(x, pos, batch, W1, b1):
    ids = batch.astype(jnp.int32)
    bounds = jnp.stack([ids[::BN], ids[BN - 1::BN]], axis=1)
    ids3 = ids.reshape(NB, BN, 1)
    wx = W1[:D]
    pos4 = jnp.concatenate([pos, jnp.ones((N, 1), dtype=pos.dtype)], axis=1)
    wp4 = jnp.concatenate([W1[D:], b1.reshape(1, 128)], axis=0)

    grid_spec = pltpu.PrefetchScalarGridSpec(
        num_scalar_prefetch=1,
        grid=(NB + NEP,),
        in_specs=[
            pl.BlockSpec((BN, D), lambda i, b_: (jnp.minimum(i, NB - 1), 0)),
            pl.BlockSpec((BN, 4), lambda i, b_: (jnp.minimum(i, NB - 1), 0)),
            pl.BlockSpec((1, BN, 1),
                         lambda i, b_: (jnp.minimum(i, NB - 1), 0, 0)),
            pl.BlockSpec((D, 128), lambda i, b_: (0, 0)),
            pl.BlockSpec((4, 128), lambda i, b_: (0, 0)),
        ],
        out_specs=pl.BlockSpec((OB, 128),
                               lambda i, b_: (jnp.maximum(i - NB, 0), 0)),
        scratch_shapes=[pltpu.VMEM((S, 128), jnp.float32)],
    )

    pooled = pl.pallas_call(
        _fused_kernel,
        grid_spec=grid_spec,
        out_shape=jax.ShapeDtypeStruct((S, 128), jnp.float32),
    )(bounds, x, pos4, ids3, wx, wp4)

    pos_out = jnp.zeros((S, 3), dtype=pos.dtype)
    batch_out = jnp.arange(S, dtype=batch.dtype)
    return pooled, pos_out, batch_out


# speedup vs baseline: 1.2147x; 1.2147x over previous
import jax
import jax.numpy as jnp
from jax.experimental import pallas as pl
from jax.experimental.pallas import tpu as pltpu

N = 320000
D = 128
S = 1024
BN = 1280
NB = N // BN
W = 4
OB = 128
NEP = S // OB


def _fused_kernel(bounds_ref, x_ref, pos_ref, ids_ref, wx_ref, wp_ref,
                  out_ref, acc_ref):
    i = pl.program_id(0)

    @pl.when(i == 0)
    def _init():
        acc_ref[...] = jnp.zeros_like(acc_ref)

    @pl.when(i < NB)
    def _compute():
        h = jnp.dot(x_ref[...], wx_ref[...],
                    preferred_element_type=jnp.float32)
        h += jnp.dot(pos_ref[...], wp_ref[...],
                     preferred_element_type=jnp.float32)

        ids = ids_ref[0]
        s_lo = bounds_ref[i, 0]
        s_hi = bounds_ref[i, 1]

        base = jnp.minimum(s_lo, S - W)
        win = jnp.concatenate(
            [jnp.max(jnp.where(ids == base + k, h, 0.0), axis=0,
                     keepdims=True) for k in range(W)], axis=0)
        cur = acc_ref[pl.ds(base, W), :]
        acc_ref[pl.ds(base, W), :] = jnp.maximum(cur, win)

        @pl.when(s_hi >= base + W)
        def _resid():
            def body(s, _):
                col = jnp.max(jnp.where(ids == s, h, 0.0), axis=0,
                              keepdims=True)
                cur = acc_ref[pl.ds(s, 1), :]
                acc_ref[pl.ds(s, 1), :] = jnp.maximum(cur, col)
                return 0

            jax.lax.fori_loop(base + W, s_hi + 1, body, 0, unroll=False)

    @pl.when(i >= NB)
    def _epilogue():
        out_ref[...] = acc_ref[pl.ds((i - NB) * OB, OB), :]


def kernel(x, pos, batch, W1, b1):
    ids = batch.astype(jnp.int32)
    bounds = jnp.stack([ids[::BN], ids[BN - 1::BN]], axis=1)
    ids3 = ids.reshape(NB, BN, 1)
    wx = W1[:D]
    pos4 = jnp.concatenate([pos, jnp.ones((N, 1), dtype=pos.dtype)], axis=1)
    wp4 = jnp.concatenate([W1[D:], b1.reshape(1, 128)], axis=0)

    grid_spec = pltpu.PrefetchScalarGridSpec(
        num_scalar_prefetch=1,
        grid=(NB + NEP,),
        in_specs=[
            pl.BlockSpec((BN, D), lambda i, b_: (jnp.minimum(i, NB - 1), 0)),
            pl.BlockSpec((BN, 4), lambda i, b_: (jnp.minimum(i, NB - 1), 0)),
            pl.BlockSpec((1, BN, 1),
                         lambda i, b_: (jnp.minimum(i, NB - 1), 0, 0)),
            pl.BlockSpec((D, 128), lambda i, b_: (0, 0)),
            pl.BlockSpec((4, 128), lambda i, b_: (0, 0)),
        ],
        out_specs=pl.BlockSpec((OB, 128),
                               lambda i, b_: (jnp.maximum(i - NB, 0), 0)),
        scratch_shapes=[pltpu.VMEM((S, 128), jnp.float32)],
    )

    pooled = pl.pallas_call(
        _fused_kernel,
        grid_spec=grid_spec,
        out_shape=jax.ShapeDtypeStruct((S, 128), jnp.float32),
    )(bounds, x, pos4, ids3, wx, wp4)

    pos_out = jnp.zeros((S, 3), dtype=pos.dtype)
    batch_out = jnp.arange(S, dtype=batch.dtype)
    return pooled, pos_out, batch_out


# speedup vs baseline: 1.8360x; 1.5114x over previous
import jax
import jax.numpy as jnp
from jax.experimental import pallas as pl
from jax.experimental.pallas import tpu as pltpu

N = 320000
D = 128
S = 1024
BN = 3200
NB = N // BN
OB = 128
NEP = S // OB
CH = 32


def _fused_kernel(starts_ref, x_ref, pos_ref, wx_ref, wp_ref,
                  out_ref, acc_ref, hbuf_ref):
    i = pl.program_id(0)

    @pl.when(i == 0)
    def _init():
        acc_ref[...] = jnp.zeros_like(acc_ref)

    @pl.when(i < NB)
    def _compute():
        h = jnp.dot(x_ref[...], wx_ref[...],
                    preferred_element_type=jnp.float32)
        h += jnp.dot(pos_ref[...], wp_ref[...],
                     preferred_element_type=jnp.float32)
        hbuf_ref[...] = h

        row0 = i * BN
        s_lo = starts_ref[S + 1 + i]
        s_hi = starts_ref[S + 1 + NB + i]
        iota = jax.lax.broadcasted_iota(jnp.int32, (CH, 1), 0)

        def seg_body(s, _):
            lo = jnp.maximum(starts_ref[s] - row0, 0)
            hi = jnp.minimum(starts_ref[s + 1] - row0, BN)

            def chunk_body(c, col8):
                r = pl.multiple_of(c * CH, CH)
                v = hbuf_ref[pl.ds(r, CH), :]
                rid = r + iota
                m = (rid >= lo) & (rid < hi)
                vm = jnp.where(m, v, 0.0)
                red = jnp.max(vm.reshape(CH // 8, 8, 128), axis=0)
                return jnp.maximum(col8, red)

            col8 = jax.lax.fori_loop(lo // CH, (hi + CH - 1) // CH,
                                     chunk_body,
                                     jnp.zeros((8, 128), jnp.float32),
                                     unroll=False)
            a = pl.multiple_of(s * 8, 8)
            acc_ref[pl.ds(a, 8), :] = jnp.maximum(acc_ref[pl.ds(a, 8), :],
                                                  col8)
            return 0

        jax.lax.fori_loop(s_lo, s_hi + 1, seg_body, 0, unroll=False)

    @pl.when(i >= NB)
    def _epilogue():
        a = acc_ref[pl.ds((i - NB) * OB * 8, OB * 8), :]
        out_ref[...] = jnp.max(a.reshape(OB, 8, 128), axis=1)


def kernel(x, pos, batch, W1, b1):
    ids = batch.astype(jnp.int32)
    seg_starts = jnp.searchsorted(
        ids, jnp.arange(S + 1, dtype=jnp.int32)).astype(jnp.int32)
    starts = jnp.concatenate([seg_starts, ids[::BN], ids[BN - 1::BN]])
    wx = W1[:D]
    pos4 = jnp.concatenate([pos, jnp.ones((N, 1), dtype=pos.dtype)], axis=1)
    wp4 = jnp.concatenate([W1[D:], b1.reshape(1, 128)], axis=0)

    grid_spec = pltpu.PrefetchScalarGridSpec(
        num_scalar_prefetch=1,
        grid=(NB + NEP,),
        in_specs=[
            pl.BlockSpec((BN, D), lambda i, b_: (jnp.minimum(i, NB - 1), 0)),
            pl.BlockSpec((BN, 4), lambda i, b_: (jnp.minimum(i, NB - 1), 0)),
            pl.BlockSpec((D, 128), lambda i, b_: (0, 0)),
            pl.BlockSpec((4, 128), lambda i, b_: (0, 0)),
        ],
        out_specs=pl.BlockSpec((OB, 128),
                               lambda i, b_: (jnp.maximum(i - NB, 0), 0)),
        scratch_shapes=[pltpu.VMEM((S * 8, 128), jnp.float32),
                        pltpu.VMEM((BN, 128), jnp.float32)],
    )

    pooled = pl.pallas_call(
        _fused_kernel,
        grid_spec=grid_spec,
        out_shape=jax.ShapeDtypeStruct((S, 128), jnp.float32),
    )(starts, x, pos4, wx, wp4)

    pos_out = jnp.zeros((S, 3), dtype=pos.dtype)
    batch_out = jnp.arange(S, dtype=batch.dtype)
    return pooled, pos_out, batch_out
